# baseline (speedup 1.0000x reference)
import functools

import jax
import jax.numpy as jnp
from jax import lax
from jax.experimental import pallas as pl
from jax.experimental.pallas import tpu as pltpu
from jax.experimental.pallas import tpu_sc as plsc

BEAM = 4
VOCAB = 32768
PAD_ID = 0
EOS_ID = 2
LPF = 0.6
BATCH = 128

ROWS = BATCH * BEAM
NW = 32
RPW = ROWS // NW
L = 16
NEG = -3.0e38
BIGIDX = 2 ** 30
NSEL = 6


def _sc_select(wf_ref, pf_ref, tf_ref, wp_ref, outk_ref, outi_ref,
               wrow, pv, tv, rowbuf, stagek, stagei):
    wid = lax.axis_index("s") * 2 + lax.axis_index("c")
    base = wid * RPW
    pltpu.sync_copy(wf_ref.at[pl.ds(base, RPW)], wrow)
    pltpu.sync_copy(pf_ref.at[pl.ds(base, RPW)], pv)
    pltpu.sync_copy(tf_ref.at[pl.ds(base, RPW)], tv)
    trips = jnp.max(tv[...])
    lane = lax.iota(jnp.int32, L)
    colmask = (lane == PAD_ID) | (lane == EOS_ID)

    def row_body(j, carry):
        idxv = jnp.full((L,), j, jnp.int32)
        rj = jnp.max(plsc.load_gather(wrow, [idxv]))
        pj = plsc.load_gather(pv, [idxv])
        pltpu.sync_copy(wp_ref.at[pl.ds(rj * VOCAB, VOCAB)], rowbuf)

        key0 = pj * rowbuf[pl.ds(0, L)]
        key0 = jnp.where(colmask, jnp.float32(-1.0), key0)
        lo = jnp.full((L,), jnp.float32(-2.0))
        zi = jnp.zeros((L,), jnp.int32)

        def chunk(k, st):
            v1, v2, v3, v4, i1, i2, i3, i4 = st
            key = pj * rowbuf[pl.ds(k * L, L)]
            ci = k
            m = key > v1
            nv1 = jnp.maximum(v1, key)
            key = jnp.minimum(v1, key)
            i1, ci = jnp.where(m, ci, i1), jnp.where(m, i1, ci)
            m = key > v2
            nv2 = jnp.maximum(v2, key)
            key = jnp.minimum(v2, key)
            i2, ci = jnp.where(m, ci, i2), jnp.where(m, i2, ci)
            m = key > v3
            nv3 = jnp.maximum(v3, key)
            key = jnp.minimum(v3, key)
            i3, ci = jnp.where(m, ci, i3), jnp.where(m, i3, ci)
            m = key > v4
            nv4 = jnp.maximum(v4, key)
            i4 = jnp.where(m, ci, i4)
            return nv1, nv2, nv3, nv4, i1, i2, i3, i4

        v1, v2, v3, v4, i1, i2, i3, i4 = lax.fori_loop(
            1, VOCAB // L, chunk,
            (key0, lo, lo, lo, zi, zi, zi, zi), unroll=8)

        kv = jnp.zeros((L,), jnp.float32)
        iv = jnp.zeros((L,), jnp.int32)
        for t in range(NSEL):
            full = i1 * L + lane
            mx = jnp.max(v1)
            eq = v1 == mx
            mi = jnp.min(jnp.where(eq, full, BIGIDX))
            sel = eq & (full == mi)
            lt = lane == t
            kv = jnp.where(lt, mx, kv)
            iv = jnp.where(lt, mi, iv)
            v1 = jnp.where(sel, v2, v1)
            i1 = jnp.where(sel, i2, i1)
            v2 = jnp.where(sel, v3, v2)
            i2 = jnp.where(sel, i3, i2)
            v3 = jnp.where(sel, v4, v3)
            i3 = jnp.where(sel, i4, i3)
            v4 = jnp.where(sel, NEG, v4)
        stagek[pl.ds(j * L, L)] = kv
        stagei[pl.ds(j * L, L)] = iv
        return carry

    lax.fori_loop(0, trips, row_body, 0)
    pltpu.sync_copy(stagek, outk_ref.at[pl.ds(base * L, RPW * L)])
    pltpu.sync_copy(stagei, outi_ref.at[pl.ds(base * L, RPW * L)])


def _sc_call(wf, pf, tf, wpf):
    mesh = plsc.VectorSubcoreMesh(core_axis_name="c", subcore_axis_name="s")
    return pl.kernel(
        _sc_select,
        mesh=mesh,
        compiler_params=pltpu.CompilerParams(needs_layout_passes=False),
        out_type=[
            jax.ShapeDtypeStruct((ROWS * L,), jnp.float32),
            jax.ShapeDtypeStruct((ROWS * L,), jnp.int32),
        ],
        scratch_types=[
            pltpu.VMEM((RPW,), jnp.int32),
            pltpu.VMEM((RPW,), jnp.float32),
            pltpu.VMEM((RPW,), jnp.int32),
            pltpu.VMEM((VOCAB,), jnp.float32),
            pltpu.VMEM((RPW * L,), jnp.float32),
            pltpu.VMEM((RPW * L,), jnp.int32),
        ],
    )(wf, pf, tf, wpf)


def _epi_kernel(p_ref, stop_ref, kc_ref, ic_ref, wp01_ref,
                score_ref, flat_ref):
    p = p_ref[...]
    stopb = stop_ref[...] != 0

    row = lax.broadcasted_iota(jnp.int32, (ROWS, 1), 0)
    beam = row % BEAM
    base = beam * VOCAB

    k = kc_ref[...]
    vi = ic_ref[...]
    lane6 = lax.broadcasted_iota(jnp.int32, (ROWS, NSEL), 1)
    stop_f = jnp.where(lane6 < BEAM, base + lane6 + 1, BIGIDX + base + lane6)
    sel_k = jnp.where(stopb, 0.0, k)
    sel_f = jnp.where(stopb, stop_f, base + vi)

    w0 = wp01_ref[:, PAD_ID:PAD_ID + 1]
    w2 = wp01_ref[:, EOS_ID:EOS_ID + 1]
    k0 = jnp.where(stopb, p, p * w0)
    f0 = base
    slot7_k = jnp.where(stopb, 0.0, p * w2)
    slot7_f = jnp.where(stopb, BIGIDX + base + 7, base + EOS_ID)

    score_ref[...] = jnp.concatenate([sel_k, k0, slot7_k], axis=1)
    flat_ref[...] = jnp.concatenate([sel_f, f0, slot7_f], axis=1)


def _merge_kernel(score_ref, flat_ref, bs_ref, nw_ref, pi_ref):
    s = score_ref[...]
    f = flat_ref[...]
    batch = lax.broadcasted_iota(jnp.int32, (BATCH, 1), 0)
    bs, nw, pi = [], [], []
    for _ in range(BEAM):
        mx = jnp.max(s, axis=1, keepdims=True)
        eq = s == mx
        mi = jnp.min(jnp.where(eq, f, jnp.int32(2 ** 31 - 1)),
                     axis=1, keepdims=True)
        sel = eq & (f == mi)
        bs.append(mx)
        nw.append(mi % VOCAB)
        pi.append(batch * BEAM + mi // VOCAB)
        s = jnp.where(sel, NEG, s)
    bs_ref[...] = jnp.concatenate(bs, axis=1)
    nw_ref[...] = jnp.concatenate(nw, axis=1)
    pi_ref[...] = jnp.concatenate(pi, axis=1)


@jax.jit
def kernel(word_prob, prob, stops, word_length):
    p2 = prob.reshape(ROWS, 1)
    st2 = stops.reshape(ROWS, 1)
    wl2 = word_length.reshape(ROWS, 1)

    perm = jnp.argsort(stops, stable=True).astype(jnp.int32)
    count = jnp.int32(ROWS) - jnp.sum(stops)
    tw = jnp.clip((count - jnp.arange(NW, dtype=jnp.int32) + (NW - 1)) // NW,
                  0, RPW)
    tf = jnp.broadcast_to(tw[:, None], (NW, RPW)).reshape(-1)
    wm = perm.reshape(RPW, NW).T
    wf = wm.reshape(-1)
    pf = prob[wf]
    outk, outi = _sc_call(wf, pf, tf, word_prob.reshape(-1))

    kc = jnp.zeros((ROWS, NSEL), jnp.float32).at[wf].set(
        outk.reshape(ROWS, L)[:, :NSEL])
    ic = jnp.zeros((ROWS, NSEL), jnp.int32).at[wf].set(
        outi.reshape(ROWS, L)[:, :NSEL])

    wph = jax.lax.slice(word_prob, (0, 0), (ROWS, 128))
    scores8, flats8 = pl.pallas_call(
        _epi_kernel,
        out_shape=[
            jax.ShapeDtypeStruct((ROWS, 8), jnp.float32),
            jax.ShapeDtypeStruct((ROWS, 8), jnp.int32),
        ],
    )(p2, st2, kc, ic, wph)

    slot_is_sel = (jnp.arange(8, dtype=jnp.int32) < NSEL).astype(jnp.int32)
    addl = slot_is_sel[None, :] * (1 - st2)
    wl_c = wl2 + addl
    lp = (jnp.power((wl_c + 5).astype(jnp.float32), LPF)
          / jnp.power(jnp.float32(6.0), LPF))
    scores = jnp.log(jnp.clip(scores8, 1e-20, 1.0)) / lp

    sc = scores.reshape(BATCH, BEAM * 8)
    fl = flats8.reshape(BATCH, BEAM * 8)
    bs, nw, pi = pl.pallas_call(
        _merge_kernel,
        out_shape=[
            jax.ShapeDtypeStruct((BATCH, BEAM), jnp.float32),
            jax.ShapeDtypeStruct((BATCH, BEAM), jnp.int32),
            jax.ShapeDtypeStruct((BATCH, BEAM), jnp.int32),
        ],
    )(sc, fl)
    return bs, nw.reshape(-1), pi.reshape(-1)


# speedup vs baseline: 1.0796x; 1.0796x over previous
import functools

import jax
import jax.numpy as jnp
from jax import lax
from jax.experimental import pallas as pl
from jax.experimental.pallas import tpu as pltpu
from jax.experimental.pallas import tpu_sc as plsc

BEAM = 4
VOCAB = 32768
PAD_ID = 0
EOS_ID = 2
LPF = 0.6
BATCH = 128

ROWS = BATCH * BEAM
NW = 32
RPW = ROWS // NW
L = 16
NEG = -3.0e38
BIGIDX = 2 ** 30
NSEL = 6


def _sc_select(wf_ref, pf_ref, tf_ref, wp_ref, outk_ref, outi_ref,
               wrow, pv, tv, rowbuf, stagek, stagei, semA, semB):
    wid = lax.axis_index("s") * 2 + lax.axis_index("c")
    base = wid * RPW
    pltpu.sync_copy(wf_ref.at[pl.ds(base, RPW)], wrow)
    pltpu.sync_copy(pf_ref.at[pl.ds(base, RPW)], pv)
    pltpu.sync_copy(tf_ref.at[pl.ds(base, RPW)], tv)
    trips = jnp.max(tv[...])
    lane = lax.iota(jnp.int32, L)
    colmask = (lane == PAD_ID) | (lane == EOS_ID)

    def rowid(j):
        return jnp.max(plsc.load_gather(wrow, [jnp.full((L,), j, jnp.int32)]))

    def dma(j, par, sem):
        return pltpu.make_async_copy(
            wp_ref.at[pl.ds(rowid(j) * VOCAB, VOCAB)],
            rowbuf.at[pl.ds(par * VOCAB, VOCAB)], sem)

    @pl.when(trips > 0)
    def _():
        dma(0, 0, semA).start()

    @pl.when(trips > 1)
    def _():
        dma(1, 1, semB).start()

    def process(j, par, carry):
        off = par * VOCAB
        idxv = jnp.full((L,), j, jnp.int32)
        pj = plsc.load_gather(pv, [idxv])

        key0 = pj * rowbuf[pl.ds(off, L)]
        key0 = jnp.where(colmask, jnp.float32(-1.0), key0)
        lo = jnp.full((L,), jnp.float32(-2.0))
        zi = jnp.zeros((L,), jnp.int32)

        def chunk(k, st):
            v1, v2, v3, v4, i1, i2, i3, i4 = st
            key = pj * rowbuf[pl.ds(off + k * L, L)]
            ci = k
            m = key > v1
            nv1 = jnp.maximum(v1, key)
            key = jnp.minimum(v1, key)
            i1, ci = jnp.where(m, ci, i1), jnp.where(m, i1, ci)
            m = key > v2
            nv2 = jnp.maximum(v2, key)
            key = jnp.minimum(v2, key)
            i2, ci = jnp.where(m, ci, i2), jnp.where(m, i2, ci)
            m = key > v3
            nv3 = jnp.maximum(v3, key)
            key = jnp.minimum(v3, key)
            i3, ci = jnp.where(m, ci, i3), jnp.where(m, i3, ci)
            m = key > v4
            nv4 = jnp.maximum(v4, key)
            i4 = jnp.where(m, ci, i4)
            return nv1, nv2, nv3, nv4, i1, i2, i3, i4

        v1, v2, v3, v4, i1, i2, i3, i4 = lax.fori_loop(
            1, VOCAB // L, chunk,
            (key0, lo, lo, lo, zi, zi, zi, zi), unroll=8)

        kv = jnp.zeros((L,), jnp.float32)
        iv = jnp.zeros((L,), jnp.int32)
        for t in range(NSEL):
            full = i1 * L + lane
            mx = jnp.max(v1)
            eq = v1 == mx
            mi = jnp.min(jnp.where(eq, full, BIGIDX))
            sel = eq & (full == mi)
            lt = lane == t
            kv = jnp.where(lt, mx, kv)
            iv = jnp.where(lt, mi, iv)
            v1 = jnp.where(sel, v2, v1)
            i1 = jnp.where(sel, i2, i1)
            v2 = jnp.where(sel, v3, v2)
            i2 = jnp.where(sel, i3, i2)
            v3 = jnp.where(sel, v4, v3)
            i3 = jnp.where(sel, i4, i3)
            v4 = jnp.where(sel, NEG, v4)
        stagek[pl.ds(j * L, L)] = kv
        stagei[pl.ds(j * L, L)] = iv
        return carry

    def pair_body(kk, carry):
        j0 = 2 * kk
        j1 = 2 * kk + 1

        @pl.when(j0 < trips)
        def _():
            dma(j0, 0, semA).wait()
            process(j0, 0, 0)

        @pl.when(j0 + 2 < trips)
        def _():
            dma(j0 + 2, 0, semA).start()

        @pl.when(j1 < trips)
        def _():
            dma(j1, 1, semB).wait()
            process(j1, 1, 0)

        @pl.when(j1 + 2 < trips)
        def _():
            dma(j1 + 2, 1, semB).start()

        return carry

    lax.fori_loop(0, RPW // 2, pair_body, 0)
    pltpu.sync_copy(stagek, outk_ref.at[pl.ds(base * L, RPW * L)])
    pltpu.sync_copy(stagei, outi_ref.at[pl.ds(base * L, RPW * L)])


def _sc_call(wf, pf, tf, wpf):
    mesh = plsc.VectorSubcoreMesh(core_axis_name="c", subcore_axis_name="s")
    return pl.kernel(
        _sc_select,
        mesh=mesh,
        compiler_params=pltpu.CompilerParams(needs_layout_passes=False),
        out_type=[
            jax.ShapeDtypeStruct((ROWS * L,), jnp.float32),
            jax.ShapeDtypeStruct((ROWS * L,), jnp.int32),
        ],
        scratch_types=[
            pltpu.VMEM((RPW,), jnp.int32),
            pltpu.VMEM((RPW,), jnp.float32),
            pltpu.VMEM((RPW,), jnp.int32),
            pltpu.VMEM((2 * VOCAB,), jnp.float32),
            pltpu.VMEM((RPW * L,), jnp.float32),
            pltpu.VMEM((RPW * L,), jnp.int32),
            pltpu.SemaphoreType.DMA,
            pltpu.SemaphoreType.DMA,
        ],
    )(wf, pf, tf, wpf)


def _epi_kernel(p_ref, stop_ref, kc_ref, ic_ref, wp01_ref,
                score_ref, flat_ref):
    p = p_ref[...]
    stopb = stop_ref[...] != 0

    row = lax.broadcasted_iota(jnp.int32, (ROWS, 1), 0)
    beam = row % BEAM
    base = beam * VOCAB

    k = kc_ref[...]
    vi = ic_ref[...]
    lane6 = lax.broadcasted_iota(jnp.int32, (ROWS, NSEL), 1)
    stop_f = jnp.where(lane6 < BEAM, base + lane6 + 1, BIGIDX + base + lane6)
    sel_k = jnp.where(stopb, 0.0, k)
    sel_f = jnp.where(stopb, stop_f, base + vi)

    w0 = wp01_ref[:, PAD_ID:PAD_ID + 1]
    w2 = wp01_ref[:, EOS_ID:EOS_ID + 1]
    k0 = jnp.where(stopb, p, p * w0)
    f0 = base
    slot7_k = jnp.where(stopb, 0.0, p * w2)
    slot7_f = jnp.where(stopb, BIGIDX + base + 7, base + EOS_ID)

    score_ref[...] = jnp.concatenate([sel_k, k0, slot7_k], axis=1)
    flat_ref[...] = jnp.concatenate([sel_f, f0, slot7_f], axis=1)


def _merge_kernel(score_ref, flat_ref, bs_ref, nw_ref, pi_ref):
    s = score_ref[...]
    f = flat_ref[...]
    batch = lax.broadcasted_iota(jnp.int32, (BATCH, 1), 0)
    bs, nw, pi = [], [], []
    for _ in range(BEAM):
        mx = jnp.max(s, axis=1, keepdims=True)
        eq = s == mx
        mi = jnp.min(jnp.where(eq, f, jnp.int32(2 ** 31 - 1)),
                     axis=1, keepdims=True)
        sel = eq & (f == mi)
        bs.append(mx)
        nw.append(mi % VOCAB)
        pi.append(batch * BEAM + mi // VOCAB)
        s = jnp.where(sel, NEG, s)
    bs_ref[...] = jnp.concatenate(bs, axis=1)
    nw_ref[...] = jnp.concatenate(nw, axis=1)
    pi_ref[...] = jnp.concatenate(pi, axis=1)


@jax.jit
def kernel(word_prob, prob, stops, word_length):
    p2 = prob.reshape(ROWS, 1)
    st2 = stops.reshape(ROWS, 1)
    wl2 = word_length.reshape(ROWS, 1)

    perm = jnp.argsort(stops, stable=True).astype(jnp.int32)
    count = jnp.int32(ROWS) - jnp.sum(stops)
    tw = jnp.clip((count - jnp.arange(NW, dtype=jnp.int32) + (NW - 1)) // NW,
                  0, RPW)
    tf = jnp.broadcast_to(tw[:, None], (NW, RPW)).reshape(-1)
    wm = perm.reshape(RPW, NW).T
    wf = wm.reshape(-1)
    pf = prob[wf]
    outk, outi = _sc_call(wf, pf, tf, word_prob.reshape(-1))

    kc = jnp.zeros((ROWS, NSEL), jnp.float32).at[wf].set(
        outk.reshape(ROWS, L)[:, :NSEL])
    ic = jnp.zeros((ROWS, NSEL), jnp.int32).at[wf].set(
        outi.reshape(ROWS, L)[:, :NSEL])

    wph = jax.lax.slice(word_prob, (0, 0), (ROWS, 128))
    scores8, flats8 = pl.pallas_call(
        _epi_kernel,
        out_shape=[
            jax.ShapeDtypeStruct((ROWS, 8), jnp.float32),
            jax.ShapeDtypeStruct((ROWS, 8), jnp.int32),
        ],
    )(p2, st2, kc, ic, wph)

    slot_is_sel = (jnp.arange(8, dtype=jnp.int32) < NSEL).astype(jnp.int32)
    addl = slot_is_sel[None, :] * (1 - st2)
    wl_c = wl2 + addl
    lp = (jnp.power((wl_c + 5).astype(jnp.float32), LPF)
          / jnp.power(jnp.float32(6.0), LPF))
    scores = jnp.log(jnp.clip(scores8, 1e-20, 1.0)) / lp

    sc = scores.reshape(BATCH, BEAM * 8)
    fl = flats8.reshape(BATCH, BEAM * 8)
    bs, nw, pi = pl.pallas_call(
        _merge_kernel,
        out_shape=[
            jax.ShapeDtypeStruct((BATCH, BEAM), jnp.float32),
            jax.ShapeDtypeStruct((BATCH, BEAM), jnp.int32),
            jax.ShapeDtypeStruct((BATCH, BEAM), jnp.int32),
        ],
    )(sc, fl)
    return bs, nw.reshape(-1), pi.reshape(-1)


# speedup vs baseline: 1.2008x; 1.1123x over previous
import functools

import jax
import jax.numpy as jnp
from jax import lax
from jax.experimental import pallas as pl
from jax.experimental.pallas import tpu as pltpu
from jax.experimental.pallas import tpu_sc as plsc

BEAM = 4
VOCAB = 32768
PAD_ID = 0
EOS_ID = 2
LPF = 0.6
BATCH = 128

ROWS = BATCH * BEAM
SC_ROWS = 128
NTC = ROWS - SC_ROWS
NW = 32
RPW = SC_ROWS // NW
L = 16
ROW_BLK = 64
RG = 32
LANES = 128
CHUNKS = VOCAB // LANES
NEG = -3.0e38
BIGIDX = 2 ** 30
NSEL = 6


def _sc_select(wf_ref, pf_ref, tf_ref, wp_ref, outk_ref, outi_ref,
               wrow, pv, tv, rowbuf, stagek, stagei):
    wid = lax.axis_index("s") * 2 + lax.axis_index("c")
    base = wid * RPW
    pltpu.sync_copy(wf_ref.at[pl.ds(wid * L, L)], wrow)
    pltpu.sync_copy(pf_ref.at[pl.ds(wid * L, L)], pv)
    pltpu.sync_copy(tf_ref.at[pl.ds(wid * L, L)], tv)
    trips = jnp.max(tv[...])
    lane = lax.iota(jnp.int32, L)
    colmask = (lane == PAD_ID) | (lane == EOS_ID)

    def row_body(j, carry):
        idxv = jnp.full((L,), j, jnp.int32)
        rj = jnp.max(plsc.load_gather(wrow, [idxv]))
        pj = plsc.load_gather(pv, [idxv])
        pltpu.sync_copy(wp_ref.at[pl.ds(rj * VOCAB, VOCAB)], rowbuf)

        key0 = pj * rowbuf[pl.ds(0, L)]
        key0 = jnp.where(colmask, jnp.float32(-1.0), key0)
        lo = jnp.full((L,), jnp.float32(-2.0))
        zi = jnp.zeros((L,), jnp.int32)

        def chunk(k, st):
            v1, v2, v3, v4, i1, i2, i3, i4 = st
            key = pj * rowbuf[pl.ds(k * L, L)]
            ci = k
            m = key > v1
            nv1 = jnp.maximum(v1, key)
            key = jnp.minimum(v1, key)
            i1, ci = jnp.where(m, ci, i1), jnp.where(m, i1, ci)
            m = key > v2
            nv2 = jnp.maximum(v2, key)
            key = jnp.minimum(v2, key)
            i2, ci = jnp.where(m, ci, i2), jnp.where(m, i2, ci)
            m = key > v3
            nv3 = jnp.maximum(v3, key)
            key = jnp.minimum(v3, key)
            i3, ci = jnp.where(m, ci, i3), jnp.where(m, i3, ci)
            m = key > v4
            nv4 = jnp.maximum(v4, key)
            i4 = jnp.where(m, ci, i4)
            return nv1, nv2, nv3, nv4, i1, i2, i3, i4

        v1, v2, v3, v4, i1, i2, i3, i4 = lax.fori_loop(
            1, VOCAB // L, chunk,
            (key0, lo, lo, lo, zi, zi, zi, zi), unroll=8)

        kv = jnp.zeros((L,), jnp.float32)
        iv = jnp.zeros((L,), jnp.int32)
        for t in range(NSEL):
            full = i1 * L + lane
            mx = jnp.max(v1)
            eq = v1 == mx
            mi = jnp.min(jnp.where(eq, full, BIGIDX))
            sel = eq & (full == mi)
            lt = lane == t
            kv = jnp.where(lt, mx, kv)
            iv = jnp.where(lt, mi, iv)
            v1 = jnp.where(sel, v2, v1)
            i1 = jnp.where(sel, i2, i1)
            v2 = jnp.where(sel, v3, v2)
            i2 = jnp.where(sel, i3, i2)
            v3 = jnp.where(sel, v4, v3)
            i3 = jnp.where(sel, i4, i3)
            v4 = jnp.where(sel, NEG, v4)
        stagek[pl.ds(j * L, L)] = kv
        stagei[pl.ds(j * L, L)] = iv
        return carry

    lax.fori_loop(0, trips, row_body, 0)
    pltpu.sync_copy(stagek, outk_ref.at[pl.ds(base * L, RPW * L)])
    pltpu.sync_copy(stagei, outi_ref.at[pl.ds(base * L, RPW * L)])


def _sc_call(wf, pf, tf, wpf):
    mesh = plsc.VectorSubcoreMesh(core_axis_name="c", subcore_axis_name="s")
    return pl.kernel(
        _sc_select,
        mesh=mesh,
        compiler_params=pltpu.CompilerParams(needs_layout_passes=False),
        out_type=[
            jax.ShapeDtypeStruct((SC_ROWS * L,), jnp.float32),
            jax.ShapeDtypeStruct((SC_ROWS * L,), jnp.int32),
        ],
        scratch_types=[
            pltpu.VMEM((L,), jnp.int32),
            pltpu.VMEM((L,), jnp.float32),
            pltpu.VMEM((L,), jnp.int32),
            pltpu.VMEM((VOCAB,), jnp.float32),
            pltpu.VMEM((RPW * L,), jnp.float32),
            pltpu.VMEM((RPW * L,), jnp.int32),
        ],
    )(wf, pf, tf, wpf)


def _sel_kernel(p_ref, stop_ref, wl_ref, wp_ref, score_ref, flat_ref,
                kscr, iscr):
    lane1 = lax.broadcasted_iota(jnp.int32, (1, LANES), 1)
    colbad = (lane1 == PAD_ID) | (lane1 == EOS_ID)

    def chain_update(key, ci, st):
        v1, v2, v3, v4, i1, i2, i3, i4 = st
        m = key > v1
        nv1 = jnp.maximum(v1, key)
        key = jnp.minimum(v1, key)
        i1, ci = jnp.where(m, ci, i1), jnp.where(m, i1, ci)
        m = key > v2
        nv2 = jnp.maximum(v2, key)
        key = jnp.minimum(v2, key)
        i2, ci = jnp.where(m, ci, i2), jnp.where(m, i2, ci)
        m = key > v3
        nv3 = jnp.maximum(v3, key)
        key = jnp.minimum(v3, key)
        i3, ci = jnp.where(m, ci, i3), jnp.where(m, i3, ci)
        m = key > v4
        nv4 = jnp.maximum(v4, key)
        i4 = jnp.where(m, ci, i4)
        return nv1, nv2, nv3, nv4, i1, i2, i3, i4

    for rg in range(ROW_BLK // RG):
        rows = pl.ds(rg * RG, RG)
        p = p_ref[rows, :]

        key0 = p * wp_ref[rows, pl.ds(0, LANES)]
        key0 = jnp.where(colbad, jnp.float32(-1.0), key0)
        lo = jnp.full((RG, LANES), jnp.float32(-2.0))
        zi = jnp.zeros((RG, LANES), jnp.int32)

        def chunk(c, carry):
            x = wp_ref[rows, pl.ds(c * LANES, LANES)]
            return chain_update(p * x, c, carry)

        st = lax.fori_loop(
            1, CHUNKS, chunk,
            (key0, lo, lo, lo, zi, zi, zi, zi),
            unroll=8)

        v1, v2, v3, v4, i1, i2, i3, i4 = st
        lanemod = lax.broadcasted_iota(jnp.int32, (RG, LANES), 1)

        for t in range(NSEL):
            full1 = i1 * LANES + lanemod
            mx = jnp.max(v1, axis=1, keepdims=True)
            eq = v1 == mx
            mi = jnp.min(jnp.where(eq, full1, BIGIDX), axis=1, keepdims=True)
            sel = eq & (full1 == mi)
            kscr[rows, t:t + 1] = mx
            iscr[rows, t:t + 1] = mi
            v1 = jnp.where(sel, v2, v1)
            i1 = jnp.where(sel, i2, i1)
            v2 = jnp.where(sel, v3, v2)
            i2 = jnp.where(sel, i3, i2)
            v3 = jnp.where(sel, v4, v3)
            i3 = jnp.where(sel, i4, i3)
            v4 = jnp.where(sel, NEG, v4)

    _epilogue(p_ref, stop_ref, kscr[...], iscr[...],
              wp_ref[:, PAD_ID:PAD_ID + 1], wp_ref[:, EOS_ID:EOS_ID + 1],
              score_ref, flat_ref, ROW_BLK)


def _epilogue(p_ref, stop_ref, k, vi, w0, w2, score_ref, flat_ref, nrows,
              row0=0):
    p = p_ref[...]
    stopb = stop_ref[...] != 0
    row = row0 + lax.broadcasted_iota(jnp.int32, (nrows, 1), 0)
    beam = row % BEAM
    base = beam * VOCAB

    lane6 = lax.broadcasted_iota(jnp.int32, (nrows, NSEL), 1)
    stop_f = jnp.where(lane6 < BEAM, base + lane6 + 1, BIGIDX + base + lane6)
    sel_k = jnp.where(stopb, 0.0, k)
    sel_f = jnp.where(stopb, stop_f, base + vi)

    k0 = jnp.where(stopb, p, p * w0)
    f0 = base
    slot7_k = jnp.where(stopb, 0.0, p * w2)
    slot7_f = jnp.where(stopb, BIGIDX + base + 7, base + EOS_ID)

    score_ref[...] = jnp.concatenate([sel_k, k0, slot7_k], axis=1)
    flat_ref[...] = jnp.concatenate([sel_f, f0, slot7_f], axis=1)


def _epi_kernel(p_ref, stop_ref, kc_ref, ic_ref, wp01_ref,
                score_ref, flat_ref):
    _epilogue(p_ref, stop_ref, kc_ref[...], ic_ref[...],
              wp01_ref[:, PAD_ID:PAD_ID + 1], wp01_ref[:, EOS_ID:EOS_ID + 1],
              score_ref, flat_ref, SC_ROWS, row0=NTC)


def _merge_kernel(score_ref, flat_ref, bs_ref, nw_ref, pi_ref):
    s = score_ref[...]
    f = flat_ref[...]
    batch = lax.broadcasted_iota(jnp.int32, (BATCH, 1), 0)
    bs, nw, pi = [], [], []
    for _ in range(BEAM):
        mx = jnp.max(s, axis=1, keepdims=True)
        eq = s == mx
        mi = jnp.min(jnp.where(eq, f, jnp.int32(2 ** 31 - 1)),
                     axis=1, keepdims=True)
        sel = eq & (f == mi)
        bs.append(mx)
        nw.append(mi % VOCAB)
        pi.append(batch * BEAM + mi // VOCAB)
        s = jnp.where(sel, NEG, s)
    bs_ref[...] = jnp.concatenate(bs, axis=1)
    nw_ref[...] = jnp.concatenate(nw, axis=1)
    pi_ref[...] = jnp.concatenate(pi, axis=1)


@jax.jit
def kernel(word_prob, prob, stops, word_length):
    p2 = prob.reshape(ROWS, 1)
    st2 = stops.reshape(ROWS, 1)
    wl2 = word_length.reshape(ROWS, 1)

    st_sc = lax.slice(stops, (NTC,), (ROWS,))
    perm = NTC + jnp.argsort(st_sc, stable=True).astype(jnp.int32)
    count = jnp.int32(SC_ROWS) - jnp.sum(st_sc)
    tw = jnp.clip((count - jnp.arange(NW, dtype=jnp.int32) + (NW - 1)) // NW,
                  0, RPW)
    tf = jnp.broadcast_to(tw[:, None], (NW, L)).reshape(-1)
    wm = perm.reshape(RPW, NW).T
    wf = wm.reshape(-1)
    wmp = jnp.pad(wm, ((0, 0), (0, L - RPW)))
    wfp = wmp.reshape(-1)
    pfp = prob[wfp]
    outk, outi = _sc_call(wfp, pfp, tf, word_prob.reshape(-1))

    grid = NTC // ROW_BLK
    keys_tc, flats_tc = pl.pallas_call(
        _sel_kernel,
        grid=(grid,),
        in_specs=[
            pl.BlockSpec((ROW_BLK, 1), lambda i: (i, 0)),
            pl.BlockSpec((ROW_BLK, 1), lambda i: (i, 0)),
            pl.BlockSpec((ROW_BLK, 1), lambda i: (i, 0)),
            pl.BlockSpec((ROW_BLK, VOCAB), lambda i: (i, 0)),
        ],
        out_specs=[
            pl.BlockSpec((ROW_BLK, 8), lambda i: (i, 0)),
            pl.BlockSpec((ROW_BLK, 8), lambda i: (i, 0)),
        ],
        out_shape=[
            jax.ShapeDtypeStruct((NTC, 8), jnp.float32),
            jax.ShapeDtypeStruct((NTC, 8), jnp.int32),
        ],
        scratch_shapes=[
            pltpu.VMEM((ROW_BLK, NSEL), jnp.float32),
            pltpu.VMEM((ROW_BLK, NSEL), jnp.int32),
        ],
    )(p2, st2, wl2, word_prob)

    wf0 = wf - NTC
    kc = jnp.zeros((SC_ROWS, NSEL), jnp.float32).at[wf0].set(
        outk.reshape(SC_ROWS, L)[:, :NSEL])
    ic = jnp.zeros((SC_ROWS, NSEL), jnp.int32).at[wf0].set(
        outi.reshape(SC_ROWS, L)[:, :NSEL])
    p_sc = lax.slice(p2, (NTC, 0), (ROWS, 1))
    st_sc2 = lax.slice(st2, (NTC, 0), (ROWS, 1))
    wph = lax.slice(word_prob, (NTC, 0), (ROWS, 128))
    keys_sc, flats_sc = pl.pallas_call(
        _epi_kernel,
        out_shape=[
            jax.ShapeDtypeStruct((SC_ROWS, 8), jnp.float32),
            jax.ShapeDtypeStruct((SC_ROWS, 8), jnp.int32),
        ],
    )(p_sc, st_sc2, kc, ic, wph)

    keys = jnp.concatenate([keys_tc, keys_sc], axis=0)
    flats = jnp.concatenate([flats_tc, flats_sc], axis=0)

    slot_is_sel = (jnp.arange(8, dtype=jnp.int32) < NSEL).astype(jnp.int32)
    addl = slot_is_sel[None, :] * (1 - st2)
    wl_c = wl2 + addl
    lp = (jnp.power((wl_c + 5).astype(jnp.float32), LPF)
          / jnp.power(jnp.float32(6.0), LPF))
    scores = jnp.log(jnp.clip(keys, 1e-20, 1.0)) / lp

    sc = scores.reshape(BATCH, BEAM * 8)
    fl = flats.reshape(BATCH, BEAM * 8)
    bs, nw, pi = pl.pallas_call(
        _merge_kernel,
        out_shape=[
            jax.ShapeDtypeStruct((BATCH, BEAM), jnp.float32),
            jax.ShapeDtypeStruct((BATCH, BEAM), jnp.int32),
            jax.ShapeDtypeStruct((BATCH, BEAM), jnp.int32),
        ],
    )(sc, fl)
    return bs, nw.reshape(-1), pi.reshape(-1)


# speedup vs baseline: 1.9845x; 1.6526x over previous
import functools

import jax
import jax.numpy as jnp
from jax.experimental import pallas as pl
from jax.experimental.pallas import tpu as pltpu

BEAM = 4
VOCAB = 32768
PAD_ID = 0
EOS_ID = 2
LPF = 0.6
BATCH = 128

ROWS = BATCH * BEAM
ROW_BLK = 64
RG = 32
LANES = 128
CHUNKS = VOCAB // LANES
NEG = -3.0e38
BIGIDX = 2 ** 30
NSEL = 6


def _sel_kernel(p_ref, stop_ref, wl_ref, wp_ref, score_ref, flat_ref,
                kscr, iscr):
    lane1 = jax.lax.broadcasted_iota(jnp.int32, (1, LANES), 1)
    colbad = (lane1 == PAD_ID) | (lane1 == EOS_ID)

    def chain_update(key, ci, st):
        v1, v2, v3, v4, i1, i2, i3, i4 = st
        m = key > v1
        nv1 = jnp.maximum(v1, key)
        key = jnp.minimum(v1, key)
        i1, ci = jnp.where(m, ci, i1), jnp.where(m, i1, ci)
        m = key > v2
        nv2 = jnp.maximum(v2, key)
        key = jnp.minimum(v2, key)
        i2, ci = jnp.where(m, ci, i2), jnp.where(m, i2, ci)
        m = key > v3
        nv3 = jnp.maximum(v3, key)
        key = jnp.minimum(v3, key)
        i3, ci = jnp.where(m, ci, i3), jnp.where(m, i3, ci)
        m = key > v4
        nv4 = jnp.maximum(v4, key)
        i4 = jnp.where(m, ci, i4)
        return nv1, nv2, nv3, nv4, i1, i2, i3, i4

    for rg in range(ROW_BLK // RG):
        rows = pl.ds(rg * RG, RG)
        p = p_ref[rows, :]

        key0 = p * wp_ref[rows, pl.ds(0, LANES)]
        key0 = jnp.where(colbad, jnp.float32(-1.0), key0)
        lo = jnp.full((RG, LANES), jnp.float32(-2.0))
        zi = jnp.zeros((RG, LANES), jnp.int32)

        def chunk(c, carry):
            x = wp_ref[rows, pl.ds(c * LANES, LANES)]
            return chain_update(p * x, c, carry)

        st = jax.lax.fori_loop(
            1, CHUNKS, chunk,
            (key0, lo, lo, lo, zi, zi, zi, zi),
            unroll=8)

        v1, v2, v3, v4, i1, i2, i3, i4 = st
        lanemod = jax.lax.broadcasted_iota(jnp.int32, (RG, LANES), 1)

        for t in range(NSEL):
            full1 = i1 * LANES + lanemod
            mx = jnp.max(v1, axis=1, keepdims=True)
            eq = v1 == mx
            mi = jnp.min(jnp.where(eq, full1, BIGIDX), axis=1, keepdims=True)
            sel = eq & (full1 == mi)
            kscr[rows, t:t + 1] = mx
            iscr[rows, t:t + 1] = mi
            v1 = jnp.where(sel, v2, v1)
            i1 = jnp.where(sel, i2, i1)
            v2 = jnp.where(sel, v3, v2)
            i2 = jnp.where(sel, i3, i2)
            v3 = jnp.where(sel, v4, v3)
            i3 = jnp.where(sel, i4, i3)
            v4 = jnp.where(sel, NEG, v4)

    p = p_ref[...]
    stopb = stop_ref[...] != 0

    row = jax.lax.broadcasted_iota(jnp.int32, (ROW_BLK, 1), 0)
    beam = row % BEAM
    base = beam * VOCAB

    k = kscr[...]
    vi = iscr[...]
    lane6 = jax.lax.broadcasted_iota(jnp.int32, (ROW_BLK, NSEL), 1)
    stop_f = jnp.where(lane6 < BEAM, base + lane6 + 1, BIGIDX + base + lane6)
    sel_k = jnp.where(stopb, 0.0, k)
    sel_f = jnp.where(stopb, stop_f, base + vi)

    w0 = wp_ref[:, PAD_ID:PAD_ID + 1]
    w2 = wp_ref[:, EOS_ID:EOS_ID + 1]
    k0 = jnp.where(stopb, p, p * w0)
    f0 = base
    slot7_k = jnp.where(stopb, 0.0, p * w2)
    slot7_f = jnp.where(stopb, BIGIDX + base + 7, base + EOS_ID)

    score_ref[...] = jnp.concatenate([sel_k, k0, slot7_k], axis=1)
    flat_ref[...] = jnp.concatenate([sel_f, f0, slot7_f], axis=1)


def _merge_kernel(score_ref, flat_ref, bs_ref, nw_ref, pi_ref):
    s = score_ref[...]
    f = flat_ref[...]
    batch = jax.lax.broadcasted_iota(jnp.int32, (BATCH, 1), 0)
    bs, nw, pi = [], [], []
    for _ in range(BEAM):
        mx = jnp.max(s, axis=1, keepdims=True)
        eq = s == mx
        mi = jnp.min(jnp.where(eq, f, jnp.int32(2 ** 31 - 1)),
                     axis=1, keepdims=True)
        sel = eq & (f == mi)
        bs.append(mx)
        nw.append(mi % VOCAB)
        pi.append(batch * BEAM + mi // VOCAB)
        s = jnp.where(sel, NEG, s)
    bs_ref[...] = jnp.concatenate(bs, axis=1)
    nw_ref[...] = jnp.concatenate(nw, axis=1)
    pi_ref[...] = jnp.concatenate(pi, axis=1)


@jax.jit
def kernel(word_prob, prob, stops, word_length):
    p2 = prob.reshape(ROWS, 1)
    st2 = stops.reshape(ROWS, 1)
    wl2 = word_length.reshape(ROWS, 1)

    grid = ROWS // ROW_BLK
    keys, flats = pl.pallas_call(
        _sel_kernel,
        grid=(grid,),
        in_specs=[
            pl.BlockSpec((ROW_BLK, 1), lambda i: (i, 0)),
            pl.BlockSpec((ROW_BLK, 1), lambda i: (i, 0)),
            pl.BlockSpec((ROW_BLK, 1), lambda i: (i, 0)),
            pl.BlockSpec((ROW_BLK, VOCAB), lambda i: (i, 0)),
        ],
        out_specs=[
            pl.BlockSpec((ROW_BLK, 8), lambda i: (i, 0)),
            pl.BlockSpec((ROW_BLK, 8), lambda i: (i, 0)),
        ],
        out_shape=[
            jax.ShapeDtypeStruct((ROWS, 8), jnp.float32),
            jax.ShapeDtypeStruct((ROWS, 8), jnp.int32),
        ],
        scratch_shapes=[
            pltpu.VMEM((ROW_BLK, NSEL), jnp.float32),
            pltpu.VMEM((ROW_BLK, NSEL), jnp.int32),
        ],
    )(p2, st2, wl2, word_prob)

    slot_is_sel = (jnp.arange(8, dtype=jnp.int32) < NSEL).astype(jnp.int32)
    addl = slot_is_sel[None, :] * (1 - st2)
    wl_c = wl2 + addl
    lp = (jnp.power((wl_c + 5).astype(jnp.float32), LPF)
          / jnp.power(jnp.float32(6.0), LPF))
    scores = jnp.log(jnp.clip(keys, 1e-20, 1.0)) / lp

    sc = scores.reshape(BATCH, BEAM * 8)
    fl = flats.reshape(BATCH, BEAM * 8)
    bs, nw, pi = pl.pallas_call(
        _merge_kernel,
        out_shape=[
            jax.ShapeDtypeStruct((BATCH, BEAM), jnp.float32),
            jax.ShapeDtypeStruct((BATCH, BEAM), jnp.int32),
            jax.ShapeDtypeStruct((BATCH, BEAM), jnp.int32),
        ],
    )(sc, fl)
    return bs, nw.reshape(-1), pi.reshape(-1)
